# baseline (speedup 1.0000x reference)
import functools

import jax
import jax.numpy as jnp
from jax import lax
from jax.experimental import pallas as pl
from jax.experimental.pallas import tpu as pltpu
from jax.experimental.pallas import tpu_sc as plsc

WALK_LEN = 50
BATCH = 4096
D = 64
C = 128
NC, NS = 2, 16
NW = NC * NS


def _sc_embed(nidx, eidx, ntab2, etab2, npos_f, epos_f):
    mesh = plsc.VectorSubcoreMesh(core_axis_name="c", subcore_axis_name="s")

    @functools.partial(
        pl.kernel,
        mesh=mesh,
        compiler_params=pltpu.CompilerParams(use_tc_tiling_on_sc=True),
        out_type=jax.ShapeDtypeStruct((2 * WALK_LEN * BATCH, D), jnp.float32),
        scratch_types=[
            pltpu.VMEM((C,), jnp.int32),
            pltpu.VMEM((C,), jnp.int32),
            pltpu.VMEM((C, 2 * D), jnp.float32),
            pltpu.VMEM((C, D), jnp.float32),
            pltpu.VMEM((WALK_LEN * D,), jnp.float32),
            pltpu.VMEM((WALK_LEN * D,), jnp.float32),
            pltpu.SemaphoreType.DMA,
        ],
    )
    def k(nidx_hbm, eidx_hbm, ntab_hbm, etab_hbm, npos_hbm, epos_hbm,
          out_hbm, idx_v, pair_v, rows_v, obuf_v, npos_v, epos_v, sem):
        wid = lax.axis_index("s") * NC + lax.axis_index("c")
        pltpu.sync_copy(npos_hbm, npos_v)
        pltpu.sync_copy(epos_hbm, epos_v)

        def do_table(idx_hbm, tab_hbm, pos_v, out_row_off):
            def body(pos, _):
                base = pos * BATCH + wid * C
                pltpu.sync_copy(idx_hbm.at[pl.ds(base, C)], idx_v)
                for g in range(C // 16):
                    sl = pl.ds(16 * g, 16)
                    pair_v[sl] = lax.shift_right_logical(idx_v[sl], 1)
                pltpu.async_copy(tab_hbm.at[pair_v], rows_v, sem).wait()
                pos_vecs = [pos_v[pl.ds(D * pos + 16 * d4, 16)]
                            for d4 in range(D // 16)]

                def grp_body(m, _):
                    halves = (idx_v[pl.ds(16 * m, 16)] & 1) * D
                    for j in range(16):
                        r = 16 * m + j
                        half = halves[j]
                        for d4 in range(D // 16):
                            obuf_v[r, pl.ds(16 * d4, 16)] = (
                                rows_v[r, pl.ds(half + 16 * d4, 16)]
                                + pos_vecs[d4])
                    return 0

                lax.fori_loop(0, C // 16, grp_body, 0)
                pltpu.sync_copy(obuf_v, out_hbm.at[pl.ds(out_row_off + base, C)])
                return 0

            lax.fori_loop(0, WALK_LEN, body, 0)

        do_table(nidx_hbm, ntab_hbm, npos_v, 0)
        do_table(eidx_hbm, etab_hbm, epos_v, WALK_LEN * BATCH)

    return k(nidx, eidx, ntab2, etab2, npos_f, epos_f)


def kernel(node_idx, edge_idx, node_table, edge_table, node_pos, edge_pos):
    nidx = node_idx.reshape(-1).astype(jnp.int32)
    eidx = edge_idx.reshape(-1).astype(jnp.int32)
    ntab2 = node_table.reshape(-1, 2 * D)
    etab2 = edge_table.reshape(-1, 2 * D)
    out = _sc_embed(nidx, eidx, ntab2, etab2,
                    node_pos.reshape(-1), edge_pos.reshape(-1))
    return out.reshape(2 * WALK_LEN, BATCH, D)
